# baseline (speedup 1.0000x reference)
import jax
import jax.numpy as jnp
import numpy as np
from jax.experimental import pallas as pl
from jax.sharding import Mesh, PartitionSpec as P

try:
    from jax import shard_map as _shard_map
except ImportError:
    from jax.experimental.shard_map import shard_map as _shard_map

TOKEN_LENGTH = 32
HIDDEN_SIZE = 128
BLOCK_ROWS = 512


def _body(x_ref, tab_ref, emb_ref, logit_ref):
    amp = x_ref[...].astype(jnp.float32) * 2.0 - 1.0
    tab = tab_ref[...]
    emb_ref[...] = amp[:, :, None] * tab[None, :, :]
    rowsum = jnp.sum(tab, axis=1)
    logit_ref[...] = amp * rowsum[None, :]


def _run(x, emb_table):
    n, L = x.shape
    H = emb_table.shape[1]
    R = min(BLOCK_ROWS, n)
    grid = (n // R,)
    return pl.pallas_call(
        _body,
        grid=grid,
        in_specs=[
            pl.BlockSpec((R, L), lambda i: (i, 0)),
            pl.BlockSpec((L, H), lambda i: (0, 0)),
        ],
        out_specs=[
            pl.BlockSpec((R, L, H), lambda i: (i, 0, 0)),
            pl.BlockSpec((R, L), lambda i: (i, 0)),
        ],
        out_shape=[
            jax.ShapeDtypeStruct((n, L, H), jnp.float32),
            jax.ShapeDtypeStruct((n, L), jnp.float32),
        ],
    )(x, emb_table)


def kernel(text_batch, emb_table):
    B, S, L = text_batch.shape
    H = emb_table.shape[1]
    N = B * S
    x = text_batch.reshape(N, L)
    devs = jax.devices()
    if len(devs) >= 2 and N % (2 * 8) == 0:
        mesh = Mesh(np.array(devs[:2]), ("d",))
        f = _shard_map(
            _run,
            mesh=mesh,
            in_specs=(P("d", None), P(None, None)),
            out_specs=(P("d", None, None), P("d", None)),
            check_vma=False,
        )
        emb_flat, logit_flat = f(x, emb_table)
    else:
        emb_flat, logit_flat = _run(x, emb_table)
    embeddings = emb_flat.reshape(B, S, L, H)
    logit_prime = logit_flat.reshape(B, S, L, 1)
    return embeddings, logit_prime


# speedup vs baseline: 1.0744x; 1.0744x over previous
import jax
import jax.numpy as jnp
import numpy as np
from jax.experimental import pallas as pl
from jax.sharding import Mesh, PartitionSpec as P

try:
    from jax import shard_map as _shard_map
except ImportError:
    from jax.experimental.shard_map import shard_map as _shard_map

TOKEN_LENGTH = 32
HIDDEN_SIZE = 128
BLOCK_ROWS = 512


def _body(x_ref, tab_ref, emb_ref, logit_ref):
    amp = x_ref[...].astype(jnp.float32) * 2.0 - 1.0
    tab = tab_ref[...]
    emb_ref[...] = amp[:, :, None] * tab[None, :, :]
    rowsum = jnp.sum(tab, axis=1)
    logit_ref[...] = amp * rowsum[None, :]


def _run(x, emb_table):
    n, L = x.shape
    H = emb_table.shape[1]
    R = min(BLOCK_ROWS, n)
    grid = (n // R,)
    return pl.pallas_call(
        _body,
        grid=grid,
        in_specs=[
            pl.BlockSpec((R, L), lambda i: (i, 0)),
            pl.BlockSpec((L, H), lambda i: (0, 0)),
        ],
        out_specs=[
            pl.BlockSpec((R, L, H), lambda i: (i, 0, 0)),
            pl.BlockSpec((R, L), lambda i: (i, 0)),
        ],
        out_shape=[
            jax.ShapeDtypeStruct((n, L, H), jnp.float32),
            jax.ShapeDtypeStruct((n, L), jnp.float32),
        ],
    )(x, emb_table)


def kernel(text_batch, emb_table):
    B, S, L = text_batch.shape
    H = emb_table.shape[1]
    N = B * S
    x = text_batch.reshape(N, L)
    devs = jax.devices()
    if len(devs) >= 2 and N % (2 * 8) == 0:
        mesh = Mesh(np.array(devs[:2]), ("d",))
        f = _shard_map(
            _run,
            mesh=mesh,
            in_specs=(P("d", None), P(None, None)),
            out_specs=(P("d", None, None), P("d", None)),
            check_vma=False,
        )
        emb_flat, logit_flat = f(x, emb_table)
        embeddings = emb_flat.reshape(B, S, L, H)
        logit_prime = logit_flat.reshape(B, S, L, 1)
        sh = jax.sharding.NamedSharding
        embeddings = jax.lax.with_sharding_constraint(
            embeddings, sh(mesh, P("d", None, None, None)))
        logit_prime = jax.lax.with_sharding_constraint(
            logit_prime, sh(mesh, P("d", None, None, None)))
        return embeddings, logit_prime
    emb_flat, logit_flat = _run(x, emb_table)
    embeddings = emb_flat.reshape(B, S, L, H)
    logit_prime = logit_flat.reshape(B, S, L, 1)
    return embeddings, logit_prime


# speedup vs baseline: 7.1021x; 6.6100x over previous
import jax
import jax.numpy as jnp
from jax.experimental import pallas as pl

TOKEN_LENGTH = 32
HIDDEN_SIZE = 128
BLOCK_ROWS = 256


def _body(x_ref, tab_ref, emb_ref, logit_ref):
    amp = x_ref[...].astype(jnp.float32) * 2.0 - 1.0
    tab = tab_ref[...]
    emb_ref[...] = amp[:, :, None] * tab[None, :, :]
    rowsum = jnp.sum(tab, axis=1)
    logit_ref[...] = amp * rowsum[None, :]


def kernel(text_batch, emb_table):
    B, S, L = text_batch.shape
    H = emb_table.shape[1]
    N = B * S
    x = text_batch.reshape(N, L)
    R = BLOCK_ROWS
    grid = (N // R,)
    emb_flat, logit_flat = pl.pallas_call(
        _body,
        grid=grid,
        in_specs=[
            pl.BlockSpec((R, L), lambda i: (i, 0)),
            pl.BlockSpec((L, H), lambda i: (0, 0)),
        ],
        out_specs=[
            pl.BlockSpec((R, L, H), lambda i: (i, 0, 0)),
            pl.BlockSpec((R, L), lambda i: (i, 0)),
        ],
        out_shape=[
            jax.ShapeDtypeStruct((N, L, H), jnp.float32),
            jax.ShapeDtypeStruct((N, L), jnp.float32),
        ],
    )(x, emb_table)
    embeddings = emb_flat.reshape(B, S, L, H)
    logit_prime = logit_flat.reshape(B, S, L, 1)
    return embeddings, logit_prime
